# Initial kernel scaffold; baseline (speedup 1.0000x reference)
import jax
import jax.numpy as jnp
from jax.experimental import pallas as pl


def kernel(node_feat, edge_feat, edge_index, batch_index, W_edge, b_edge, W_root, b_root, bn_gamma, bn_beta):
    raise NotImplementedError("write your pallas kernel here")


# speedup vs baseline: 5.2708x; 5.2708x over previous
import functools

import jax
import jax.numpy as jnp
from jax import lax
from jax.experimental import pallas as pl
from jax.experimental.pallas import tpu as pltpu
from jax.experimental.pallas import tpu_sc as plsc

N = 50000
E = 800000
IN_DIM = 16
OUT_DIM = 16
EDGE_DIM = 16

NC = 2
NS = 16
NW = NC * NS

IROWS = 6400
ICHUNK = 125
WROWS = IROWS // NW
BROWS = 8
NBLK = WROWS // BROWS

NROWS_PER_SUB = N // NS

BE = 3200

_sc_mesh = plsc.VectorSubcoreMesh(core_axis_name="c", subcore_axis_name="s")


def _gather_body(node_hbm, src_hbm, xj_hbm, idx_v, rows_v, sem):
    wid = lax.axis_index("s") * NC + lax.axis_index("c")
    base = wid * WROWS

    def blk(j, _):
        row0 = base + j * BROWS
        pltpu.sync_copy(src_hbm.at[pl.ds(row0, BROWS)], idx_v)
        copies = [
            pltpu.async_copy(node_hbm.at[idx_v.at[jj]], rows_v.at[jj], sem)
            for jj in range(BROWS)
        ]
        for cp in copies:
            cp.wait()
        pltpu.sync_copy(rows_v, xj_hbm.at[pl.ds(row0, BROWS)])
        return _

    lax.fori_loop(0, NBLK, blk, None)


_gather = pl.kernel(
    _gather_body,
    out_type=jax.ShapeDtypeStruct((IROWS, ICHUNK, IN_DIM), jnp.float32),
    mesh=_sc_mesh,
    compiler_params=pltpu.CompilerParams(use_tc_tiling_on_sc=False),
    scratch_types=[
        pltpu.VMEM((BROWS, ICHUNK), jnp.int32),
        pltpu.VMEM((BROWS, ICHUNK, IN_DIM), jnp.float32),
        pltpu.SemaphoreType.DMA,
    ],
)


def _scatter_msgs_body(msgs_hbm, dst_hbm, zeros_hbm, sums_hbm,
                       idx_v, msg_v, node_v, acc):
    cid = lax.axis_index("c")
    sid = lax.axis_index("s")
    wid = sid * NC + cid
    base = wid * WROWS
    nrow0 = sid * NROWS_PER_SUB

    pltpu.sync_copy(zeros_hbm, node_v)
    pltpu.sync_copy(node_v, acc.at[pl.ds(nrow0, NROWS_PER_SUB)])
    plsc.subcore_barrier()

    def blk(j, _):
        row0 = base + j * BROWS
        pltpu.sync_copy(dst_hbm.at[pl.ds(row0, BROWS)], idx_v)
        pltpu.sync_copy(msgs_hbm.at[pl.ds(row0, BROWS)], msg_v)
        for jj in range(BROWS):
            pltpu.sync_copy(msg_v.at[jj], acc.at[idx_v.at[jj]], add=True)
        return _

    lax.fori_loop(0, NBLK, blk, None)
    plsc.subcore_barrier()

    pltpu.sync_copy(acc.at[pl.ds(nrow0, NROWS_PER_SUB)], node_v)
    pltpu.sync_copy(node_v, sums_hbm.at[cid, pl.ds(nrow0, NROWS_PER_SUB)])


_scatter_msgs = pl.kernel(
    _scatter_msgs_body,
    out_type=jax.ShapeDtypeStruct((NC, N, OUT_DIM), jnp.float32),
    mesh=_sc_mesh,
    compiler_params=pltpu.CompilerParams(use_tc_tiling_on_sc=False),
    scratch_types=[
        pltpu.VMEM((BROWS, ICHUNK), jnp.int32),
        pltpu.VMEM((BROWS, ICHUNK, OUT_DIM), jnp.float32),
        pltpu.VMEM((NROWS_PER_SUB, OUT_DIM), jnp.float32),
        pltpu.VMEM_SHARED((N, OUT_DIM), jnp.float32),
    ],
)


def _scatter_ones_body(dst_hbm, ones_hbm, zeros_hbm, cnts_hbm,
                       idx_v, ones_v, node_v, acc):
    cid = lax.axis_index("c")
    sid = lax.axis_index("s")
    wid = sid * NC + cid
    base = wid * WROWS
    nrow0 = sid * NROWS_PER_SUB

    pltpu.sync_copy(zeros_hbm, node_v)
    pltpu.sync_copy(node_v, acc.at[pl.ds(nrow0, NROWS_PER_SUB)])
    pltpu.sync_copy(ones_hbm, ones_v)
    plsc.subcore_barrier()

    def blk(j, _):
        row0 = base + j * BROWS
        pltpu.sync_copy(dst_hbm.at[pl.ds(row0, BROWS)], idx_v)
        for jj in range(BROWS):
            pltpu.sync_copy(ones_v, acc.at[idx_v.at[jj]], add=True)
        return _

    lax.fori_loop(0, NBLK, blk, None)
    plsc.subcore_barrier()

    pltpu.sync_copy(acc.at[pl.ds(nrow0, NROWS_PER_SUB)], node_v)
    pltpu.sync_copy(node_v, cnts_hbm.at[cid, pl.ds(nrow0, NROWS_PER_SUB)])


_scatter_ones = pl.kernel(
    _scatter_ones_body,
    out_type=jax.ShapeDtypeStruct((NC, N, OUT_DIM), jnp.float32),
    mesh=_sc_mesh,
    compiler_params=pltpu.CompilerParams(use_tc_tiling_on_sc=False),
    scratch_types=[
        pltpu.VMEM((BROWS, ICHUNK), jnp.int32),
        pltpu.VMEM((ICHUNK, OUT_DIM), jnp.float32),
        pltpu.VMEM((NROWS_PER_SUB, OUT_DIM), jnp.float32),
        pltpu.VMEM_SHARED((N, OUT_DIM), jnp.float32),
    ],
)


def _dense_body(eft_ref, xjt_ref, wt_ref, bt_ref, out_ref):
    t = jnp.tanh(jnp.dot(wt_ref[...], eft_ref[...],
                         preferred_element_type=jnp.float32) + bt_ref[...])
    acc = xjt_ref[0:1, :] * t[0:OUT_DIM, :]
    for i in range(1, IN_DIM):
        acc = acc + xjt_ref[i:i + 1, :] * t[i * OUT_DIM:(i + 1) * OUT_DIM, :]
    out_ref[...] = acc


def _dense(eft, xjt, wt, bt):
    grid = (E // BE,)
    return pl.pallas_call(
        _dense_body,
        grid=grid,
        in_specs=[
            pl.BlockSpec((EDGE_DIM, BE), lambda i: (0, i)),
            pl.BlockSpec((IN_DIM, BE), lambda i: (0, i)),
            pl.BlockSpec((IN_DIM * OUT_DIM, EDGE_DIM), lambda i: (0, 0)),
            pl.BlockSpec((IN_DIM * OUT_DIM, 1), lambda i: (0, 0)),
        ],
        out_specs=pl.BlockSpec((OUT_DIM, BE), lambda i: (0, i)),
        out_shape=jax.ShapeDtypeStruct((OUT_DIM, E), jnp.float32),
    )(eft, xjt, wt, bt)


def _finish_body(sums_ref, cnts_ref, node_ref, wbig_ref, bbig_ref,
                 gbig_ref, betab_ref, fold_ref, unfold_ref, out_ref):
    s = sums_ref[0] + sums_ref[1]
    c = cnts_ref[0] + cnts_ref[1]
    aggr = s / jnp.maximum(c, 1.0)
    root = jnp.dot(node_ref[...], wbig_ref[...],
                   preferred_element_type=jnp.float32,
                   precision=lax.Precision.HIGHEST)
    pre = aggr + root + bbig_ref[...]
    colsum = jnp.sum(pre, axis=0, keepdims=True)
    colsq = jnp.sum(pre * pre, axis=0, keepdims=True)
    tot = jnp.dot(colsum, fold_ref[...], preferred_element_type=jnp.float32,
                  precision=lax.Precision.HIGHEST)
    totsq = jnp.dot(colsq, fold_ref[...], preferred_element_type=jnp.float32,
                    precision=lax.Precision.HIGHEST)
    mean16 = tot / float(N)
    var16 = totsq / float(N) - mean16 * mean16
    mean_b = jnp.dot(mean16, unfold_ref[...],
                     preferred_element_type=jnp.float32,
                     precision=lax.Precision.HIGHEST)
    var_b = jnp.dot(var16, unfold_ref[...],
                    preferred_element_type=jnp.float32,
                    precision=lax.Precision.HIGHEST)
    y = (pre - mean_b) * lax.rsqrt(var_b + 1e-5) * gbig_ref[...] \
        + betab_ref[...]
    out_ref[...] = jnp.where(y >= 0.0, y, 0.01 * y)


def _finish(sums_r, cnts_r, node_r, wbig, bbig, gbig, betab, fold, unfold):
    nr = N // IN_DIM
    lanes = IN_DIM * OUT_DIM
    return pl.pallas_call(
        _finish_body,
        out_shape=jax.ShapeDtypeStruct((nr, lanes), jnp.float32),
    )(sums_r, cnts_r, node_r, wbig, bbig, gbig, betab, fold, unfold)


def kernel(node_feat, edge_feat, edge_index, batch_index,
           W_edge, b_edge, W_root, b_root, bn_gamma, bn_beta):
    del batch_index
    src = edge_index[0].reshape(IROWS, ICHUNK).astype(jnp.int32)
    dst = edge_index[1].reshape(IROWS, ICHUNK).astype(jnp.int32)

    xj3 = _gather(node_feat, src)
    xjt = xj3.reshape(E, IN_DIM).T

    eft = edge_feat.T
    wt = W_edge.T
    bt = b_edge.reshape(IN_DIM * OUT_DIM, 1)
    msgst = _dense(eft, xjt, wt, bt)
    msgs3 = msgst.T.reshape(IROWS, ICHUNK, OUT_DIM)

    ones_rows = jnp.ones((ICHUNK, OUT_DIM), jnp.float32)
    zeros_rows = jnp.zeros((NROWS_PER_SUB, OUT_DIM), jnp.float32)
    sums = _scatter_msgs(msgs3, dst, zeros_rows)
    cnts = _scatter_ones(dst, ones_rows, zeros_rows)

    eye = jnp.eye(IN_DIM, dtype=jnp.float32)
    wbig = jnp.kron(eye, W_root)
    fold = jnp.kron(jnp.ones((IN_DIM, 1), jnp.float32), eye)
    unfold = fold.T
    bbig = jnp.tile(b_root, IN_DIM).reshape(1, IN_DIM * OUT_DIM)
    gbig = jnp.tile(bn_gamma, IN_DIM).reshape(1, IN_DIM * OUT_DIM)
    betab = jnp.tile(bn_beta, IN_DIM).reshape(1, IN_DIM * OUT_DIM)

    nr = N // IN_DIM
    lanes = IN_DIM * OUT_DIM
    out_r = _finish(
        sums.reshape(NC, nr, lanes), cnts.reshape(NC, nr, lanes),
        node_feat.reshape(nr, lanes), wbig, bbig, gbig, betab, fold, unfold)
    return out_r.reshape(N, OUT_DIM)
